# baseline (speedup 1.0000x reference)
import jax
import jax.numpy as jnp
from jax import lax
from jax.experimental import pallas as pl
from jax.experimental.pallas import tpu as pltpu
from jax.experimental.pallas import tpu_sc as plsc

B = 16384
F = 32
NC = 2
NS = 16
L = 16
NW = NC * NS
BPW = B // NW
CHUNK = 128
NCHUNK = BPW // CHUNK
GROUPS = BPW // L


def _gmfd_body(user, item, ue_t, ie_t, h_w, h_b, out_hbm,
               uidx, iidx, urt, irt, wv, bv, outv, sem):
    wid = lax.axis_index("s") * NC + lax.axis_index("c")
    base = wid * BPW

    cu = pltpu.async_copy(user.at[pl.ds(wid * NCHUNK, NCHUNK)], uidx, sem)
    ci = pltpu.async_copy(item.at[pl.ds(wid * NCHUNK, NCHUNK)], iidx, sem)
    pltpu.sync_copy(h_w, wv)
    pltpu.sync_copy(h_b, bv)
    cu.wait()
    ci.wait()

    def fire(f, _):
        for j in range(NCHUNK):
            pltpu.async_copy(ue_t.at[f].at[uidx.at[j]],
                             urt.at[f, pl.ds(j * CHUNK, CHUNK)], sem)
            pltpu.async_copy(ie_t.at[f].at[iidx.at[j]],
                             irt.at[f, pl.ds(j * CHUNK, CHUNK)], sem)
        return 0

    lax.fori_loop(0, F, fire, 0)

    pltpu.make_async_copy(ue_t.at[:, pl.ds(0, BPW)], urt, sem).wait()
    pltpu.make_async_copy(ie_t.at[:, pl.ds(0, BPW)], irt, sem).wait()

    w_lo = wv[pl.ds(0, L)]
    w_hi = wv[pl.ds(L, L)]
    bb = bv[...]

    def gbody(g, _):
        s0 = g * L
        acc = jnp.zeros((L,), jnp.float32)
        for f in range(F):
            wb = (w_lo if f < L else w_hi).at[
                jnp.full((L,), f % L, jnp.int32)].get(mode="promise_in_bounds")
            acc = acc + urt[f, pl.ds(s0, L)] * irt[f, pl.ds(s0, L)] * wb
        x = acc + bb
        outv[pl.ds(s0, L)] = 1.0 / (1.0 + jnp.exp(-x))
        return 0

    lax.fori_loop(0, GROUPS, gbody, 0)

    pltpu.sync_copy(outv, out_hbm.at[pl.ds(base, BPW)])


def kernel(user, item, user_emb, item_emb, h_w, h_b):
    ue_t = user_emb.T
    ie_t = item_emb.T
    w_flat = h_w.reshape(F)
    b_bcast = jnp.broadcast_to(h_b, (L,))
    k = pl.kernel(
        _gmfd_body,
        out_type=jax.ShapeDtypeStruct((B,), jnp.float32),
        mesh=plsc.VectorSubcoreMesh(core_axis_name="c", subcore_axis_name="s"),
        compiler_params=pltpu.CompilerParams(use_tc_tiling_on_sc=False),
        scratch_types=[
            pltpu.VMEM((NCHUNK, CHUNK), jnp.int32),
            pltpu.VMEM((NCHUNK, CHUNK), jnp.int32),
            pltpu.VMEM((F, BPW), jnp.float32),
            pltpu.VMEM((F, BPW), jnp.float32),
            pltpu.VMEM((F,), jnp.float32),
            pltpu.VMEM((L,), jnp.float32),
            pltpu.VMEM((BPW,), jnp.float32),
            pltpu.SemaphoreType.DMA,
        ],
    )
    user_r = user.astype(jnp.int32).reshape(NW * NCHUNK, CHUNK)
    item_r = item.astype(jnp.int32).reshape(NW * NCHUNK, CHUNK)
    return k(user_r, item_r, ue_t, ie_t, w_flat, b_bcast)
